# baseline (speedup 1.0000x reference)
import functools

import jax
import jax.numpy as jnp
from jax import lax
from jax.experimental import pallas as pl
from jax.experimental.pallas import tpu as pltpu
from jax.experimental.pallas import tpu_sc as plsc

_NUM_BOXES = 20000
_NUM_CLASSES = 80
_NUM_SEL = 20000


_BLK = _NUM_BOXES
_GRID = (_NUM_BOXES + _BLK - 1) // _BLK


def _reduce_body(s_ref, maxv_ref, cls_ref):
    s = s_ref[...]
    m = jnp.max(s, axis=0, keepdims=True)
    ids = lax.broadcasted_iota(jnp.int32, s.shape, 0)
    cl = jnp.min(jnp.where(s == m, ids, _NUM_CLASSES), axis=0, keepdims=True)
    maxv_ref[...] = m[0]
    cls_ref[...] = cl.astype(jnp.float32)[0]


def _cls_reduce(scores2d):
    return pl.pallas_call(
        _reduce_body,
        grid=(_GRID,),
        in_specs=[pl.BlockSpec((_NUM_CLASSES, _BLK), lambda i: (0, i))],
        out_specs=[
            pl.BlockSpec((_BLK,), lambda i: (i,)),
            pl.BlockSpec((_BLK,), lambda i: (i,)),
        ],
        out_shape=[
            jax.ShapeDtypeStruct((_NUM_BOXES,), jnp.float32),
            jax.ShapeDtypeStruct((_NUM_BOXES,), jnp.float32),
        ],
    )(scores2d)


_NUM_SC_CORES = 2
_NUM_SC_SUBCORES = 16
_NW = _NUM_SC_CORES * _NUM_SC_SUBCORES
_CHW = 624
_TAIL_BASE = _NW * _CHW
_TAIL = _NUM_SEL - _TAIL_BASE

_mesh = plsc.VectorSubcoreMesh(
    core_axis_name="c", subcore_axis_name="s",
    num_cores=_NUM_SC_CORES, num_subcores=_NUM_SC_SUBCORES,
)

def _make_sc_gather(nsrc):
    @functools.partial(
        pl.kernel,
        out_type=jax.ShapeDtypeStruct((nsrc, _NUM_SEL), jnp.float32),
        mesh=_mesh,
        scratch_types=[
            pltpu.VMEM((_CHW,), jnp.int32),
            pltpu.VMEM((nsrc, _CHW), jnp.float32),
            pltpu.VMEM((_TAIL,), jnp.int32),
            pltpu.VMEM((nsrc, _TAIL), jnp.float32),
            pltpu.SemaphoreType.DMA,
        ],
        compiler_params=pltpu.CompilerParams(
            use_tc_tiling_on_sc=False, needs_layout_passes=False),
    )
    def _gather(*refs):
        srcs = refs[:nsrc]
        idx_hbm = refs[nsrc]
        out_hbm = refs[nsrc + 1]
        idx_v, cols_v, tidx_v, tcols_v, sem = refs[nsrc + 2:]
        wid = lax.axis_index("s") * _NUM_SC_CORES + lax.axis_index("c")

        def gather_cols(n, base, idx_ref, cols_ref):
            pltpu.sync_copy(idx_hbm.at[pl.ds(base, n)], idx_ref)
            copies = [pltpu.async_copy(src.at[idx_ref], cols_ref.at[j], sem)
                      for j, src in enumerate(srcs)]
            for c in copies:
                c.wait()
            for j in range(nsrc):
                pltpu.sync_copy(cols_ref.at[j], out_hbm.at[j, pl.ds(base, n)])

        gather_cols(_CHW, wid * _CHW, idx_v, cols_v)

        @pl.when(wid == 0)
        def _tail():
            gather_cols(_TAIL, _TAIL_BASE, tidx_v, tcols_v)

    return _gather


_sc_gather_boxes = _make_sc_gather(4)
_sc_gather_scores = _make_sc_gather(2)


def kernel(idxTensor, boxes, scores):
    idx = idxTensor[:, 2]
    out4 = _sc_gather_boxes(
        boxes[0, 0], boxes[0, 1], boxes[0, 2], boxes[0, 3], idx)
    maxv, clsf = _cls_reduce(scores[0])
    out2 = _sc_gather_scores(maxv, clsf, idx)
    concatenated = jnp.concatenate([out4, out2], axis=0).T
    batches = idxTensor[:, 0]
    return (concatenated, batches)


# speedup vs baseline: 1.1401x; 1.1401x over previous
import functools

import jax
import jax.numpy as jnp
from jax import lax
from jax.experimental import pallas as pl
from jax.experimental.pallas import tpu as pltpu
from jax.experimental.pallas import tpu_sc as plsc

_NUM_BOXES = 20000
_NUM_CLASSES = 80
_NUM_SEL = 20000


_HALF = 10240
_PAD = 2 * _HALF


def _reduce_body(s1_ref, s2_ref, maxv_ref, cls_ref):
    s = jnp.concatenate([s1_ref[...], s2_ref[...]], axis=1)
    m = jnp.max(s, axis=0, keepdims=True)
    ids = lax.broadcasted_iota(jnp.int32, s.shape, 0)
    cl = jnp.min(jnp.where(s == m, ids, _NUM_CLASSES), axis=0, keepdims=True)
    maxv_ref[...] = m[0]
    cls_ref[...] = cl.astype(jnp.float32)[0]


def _cls_reduce(scores2d):
    return pl.pallas_call(
        _reduce_body,
        grid=(1,),
        in_specs=[
            pl.BlockSpec((_NUM_CLASSES, _HALF), lambda i: (0, 0)),
            pl.BlockSpec((_NUM_CLASSES, _HALF), lambda i: (0, 1)),
        ],
        out_specs=[
            pl.BlockSpec((_PAD,), lambda i: (0,)),
            pl.BlockSpec((_PAD,), lambda i: (0,)),
        ],
        out_shape=[
            jax.ShapeDtypeStruct((_PAD,), jnp.float32),
            jax.ShapeDtypeStruct((_PAD,), jnp.float32),
        ],
    )(scores2d, scores2d)


_NUM_SC_CORES = 2
_NUM_SC_SUBCORES = 16
_NW = _NUM_SC_CORES * _NUM_SC_SUBCORES
_CHW = 624
_TAIL_BASE = _NW * _CHW
_TAIL = _NUM_SEL - _TAIL_BASE

_mesh = plsc.VectorSubcoreMesh(
    core_axis_name="c", subcore_axis_name="s",
    num_cores=_NUM_SC_CORES, num_subcores=_NUM_SC_SUBCORES,
)

def _make_sc_gather(nsrc):
    @functools.partial(
        pl.kernel,
        out_type=jax.ShapeDtypeStruct((nsrc, _NUM_SEL), jnp.float32),
        mesh=_mesh,
        scratch_types=[
            pltpu.VMEM((_CHW,), jnp.int32),
            pltpu.VMEM((nsrc, _CHW), jnp.float32),
            pltpu.VMEM((_TAIL,), jnp.int32),
            pltpu.VMEM((nsrc, _TAIL), jnp.float32),
            pltpu.SemaphoreType.DMA,
        ],
        compiler_params=pltpu.CompilerParams(
            use_tc_tiling_on_sc=False, needs_layout_passes=False),
    )
    def _gather(*refs):
        srcs = refs[:nsrc]
        idx_hbm = refs[nsrc]
        out_hbm = refs[nsrc + 1]
        idx_v, cols_v, tidx_v, tcols_v, sem = refs[nsrc + 2:]
        wid = lax.axis_index("s") * _NUM_SC_CORES + lax.axis_index("c")

        def gather_cols(n, base, idx_ref, cols_ref):
            pltpu.sync_copy(idx_hbm.at[pl.ds(base, n)], idx_ref)
            copies = [pltpu.async_copy(src.at[idx_ref], cols_ref.at[j], sem)
                      for j, src in enumerate(srcs)]
            for c in copies:
                c.wait()
            for j in range(nsrc):
                pltpu.sync_copy(cols_ref.at[j], out_hbm.at[j, pl.ds(base, n)])

        gather_cols(_CHW, wid * _CHW, idx_v, cols_v)

        @pl.when(wid == 0)
        def _tail():
            gather_cols(_TAIL, _TAIL_BASE, tidx_v, tcols_v)

    return _gather


_sc_gather6 = _make_sc_gather(6)


def kernel(idxTensor, boxes, scores):
    idx = idxTensor[:, 2]
    maxv, clsf = _cls_reduce(scores[0])
    out6 = _sc_gather6(
        boxes[0, 0], boxes[0, 1], boxes[0, 2], boxes[0, 3],
        maxv, clsf, idx)
    concatenated = out6.T
    batches = idxTensor[:, 0]
    return (concatenated, batches)


# speedup vs baseline: 1.1585x; 1.0161x over previous
import functools

import jax
import jax.numpy as jnp
from jax import lax
from jax.experimental import pallas as pl
from jax.experimental.pallas import tpu as pltpu
from jax.experimental.pallas import tpu_sc as plsc

_NUM_BOXES = 20000
_NUM_CLASSES = 80
_NUM_SEL = 20000


def _reduce_body(s_ref, maxv_ref, cls_ref):
    s = s_ref[...]
    m = jnp.max(s, axis=0, keepdims=True)
    ids = lax.broadcasted_iota(jnp.int32, s.shape, 0)
    cl = jnp.min(jnp.where(s == m, ids, _NUM_CLASSES), axis=0, keepdims=True)
    maxv_ref[...] = m[0]
    cls_ref[...] = cl.astype(jnp.float32)[0]


def _cls_reduce(scores2d):
    return pl.pallas_call(
        _reduce_body,
        grid=(1,),
        in_specs=[pl.BlockSpec((_NUM_CLASSES, _NUM_BOXES), lambda i: (0, 0))],
        out_specs=[
            pl.BlockSpec((_NUM_BOXES,), lambda i: (0,)),
            pl.BlockSpec((_NUM_BOXES,), lambda i: (0,)),
        ],
        out_shape=[
            jax.ShapeDtypeStruct((_NUM_BOXES,), jnp.float32),
            jax.ShapeDtypeStruct((_NUM_BOXES,), jnp.float32),
        ],
    )(scores2d)


_NUM_SC_CORES = 2
_NUM_SC_SUBCORES = 16
_NW = _NUM_SC_CORES * _NUM_SC_SUBCORES
_CHW = 624
_TAIL_BASE = _NW * _CHW
_TAIL = _NUM_SEL - _TAIL_BASE

_mesh = plsc.VectorSubcoreMesh(
    core_axis_name="c", subcore_axis_name="s",
    num_cores=_NUM_SC_CORES, num_subcores=_NUM_SC_SUBCORES,
)

def _make_sc_gather(nsrc):
    @functools.partial(
        pl.kernel,
        out_type=jax.ShapeDtypeStruct((nsrc, _NUM_SEL), jnp.float32),
        mesh=_mesh,
        scratch_types=[
            pltpu.VMEM((_CHW,), jnp.int32),
            pltpu.VMEM((nsrc, _CHW), jnp.float32),
            pltpu.VMEM((_TAIL,), jnp.int32),
            pltpu.VMEM((nsrc, _TAIL), jnp.float32),
            pltpu.SemaphoreType.DMA,
        ],
        compiler_params=pltpu.CompilerParams(
            use_tc_tiling_on_sc=False, needs_layout_passes=False),
    )
    def _gather(*refs):
        srcs = refs[:nsrc]
        idx_hbm = refs[nsrc]
        out_hbm = refs[nsrc + 1]
        idx_v, cols_v, tidx_v, tcols_v, sem = refs[nsrc + 2:]
        wid = lax.axis_index("s") * _NUM_SC_CORES + lax.axis_index("c")

        def gather_cols(n, base, idx_ref, cols_ref):
            pltpu.sync_copy(idx_hbm.at[pl.ds(base, n)], idx_ref)
            copies = [pltpu.async_copy(src.at[idx_ref], cols_ref.at[j], sem)
                      for j, src in enumerate(srcs)]
            for c in copies:
                c.wait()
            pltpu.sync_copy(cols_ref, out_hbm.at[:, pl.ds(base, n)])

        gather_cols(_CHW, wid * _CHW, idx_v, cols_v)

        @pl.when(wid == 0)
        def _tail():
            gather_cols(_TAIL, _TAIL_BASE, tidx_v, tcols_v)

    return _gather


_sc_gather6 = _make_sc_gather(6)


def kernel(idxTensor, boxes, scores):
    idx = idxTensor[:, 2]
    maxv, clsf = _cls_reduce(scores[0])
    out6 = _sc_gather6(
        boxes[0, 0], boxes[0, 1], boxes[0, 2], boxes[0, 3],
        maxv, clsf, idx)
    concatenated = out6.T
    batches = idxTensor[:, 0]
    return (concatenated, batches)
